# baseline (speedup 1.0000x reference)
import jax
import jax.numpy as jnp
from jax import lax
from jax.experimental import pallas as pl
from jax.experimental.pallas import tpu as pltpu

_CHUNK = 8
_NBUF = 8


def _pipeline_body(x_hbm, t_ref, o_hbm, in_buf, out_buf, in_sem, out_sem):
    n_chunks = x_hbm.shape[0] // _CHUNK
    nbuf = min(_NBUF, n_chunks)

    def in_copy(c, slot):
        return pltpu.make_async_copy(
            x_hbm.at[pl.ds(c * _CHUNK, _CHUNK)],
            in_buf.at[slot],
            in_sem.at[slot],
        )

    def out_copy(c, slot):
        return pltpu.make_async_copy(
            out_buf.at[slot],
            o_hbm.at[pl.ds(c * _CHUNK, _CHUNK)],
            out_sem.at[slot],
        )

    for k in range(nbuf):
        in_copy(k, k).start()

    def step(i, carry):
        slot = lax.rem(i, nbuf)

        @pl.when(i >= nbuf)
        def _():
            out_copy(i - nbuf, slot).wait()

        in_copy(i, slot).wait()
        out_buf[slot] = in_buf[slot] + t_ref[...]
        out_copy(i, slot).start()

        @pl.when(i + nbuf < n_chunks)
        def _():
            in_copy(i + nbuf, slot).start()

        return carry

    lax.fori_loop(0, n_chunks, step, 0)

    for k in range(nbuf):
        out_copy(0, k).wait()


def kernel(patches, pos_table):
    b, n, d = patches.shape
    x_t = jnp.transpose(patches, (0, 2, 1))
    t_t = jnp.transpose(pos_table, (1, 0))
    out_t = pl.pallas_call(
        _pipeline_body,
        in_specs=[
            pl.BlockSpec(memory_space=pl.ANY),
            pl.BlockSpec(memory_space=pltpu.MemorySpace.VMEM),
        ],
        out_specs=pl.BlockSpec(memory_space=pl.ANY),
        out_shape=jax.ShapeDtypeStruct((b, d, n), patches.dtype),
        scratch_shapes=[
            pltpu.VMEM((_NBUF, _CHUNK, d, n), patches.dtype),
            pltpu.VMEM((_NBUF, _CHUNK, d, n), patches.dtype),
            pltpu.SemaphoreType.DMA((_NBUF,)),
            pltpu.SemaphoreType.DMA((_NBUF,)),
        ],
    )(x_t, t_t.reshape(1, d, n))
    return jnp.transpose(out_t, (0, 2, 1))
